# baseline (speedup 1.0000x reference)
import functools

import jax
import jax.numpy as jnp
from jax import lax
from jax.experimental import pallas as pl
from jax.experimental.pallas import tpu as pltpu
from jax.experimental.pallas import tpu_sc as plsc

N_ROWS = 100000
N_FEAT = 128
N_SEG = 512
NC = 2
NS = 16
COLS = N_FEAT // NC
NQ = COLS // 16
SEG_PER_TILE = N_SEG // NS
CHUNK = 128
GROUP = 16
NBUF = 4
ROWS_MAIN = 6256
ROWS_LAST = N_ROWS - (NS - 1) * ROWS_MAIN
NFULL = ROWS_LAST // CHUNK
REM_MAIN = ROWS_MAIN - NFULL * CHUNK
REM_LAST = ROWS_LAST - NFULL * CHUNK

_mesh = plsc.VectorSubcoreMesh(core_axis_name="c", subcore_axis_name="s")


def _tree_sum(vals):
    vals = list(vals)
    while len(vals) > 1:
        nxt = [vals[i] + vals[i + 1] for i in range(0, len(vals) - 1, 2)]
        if len(vals) % 2:
            nxt.append(vals[-1])
        vals = nxt
    return vals[0]


@functools.partial(
    pl.kernel,
    out_type=jax.ShapeDtypeStruct((N_SEG, N_FEAT), jnp.float32),
    mesh=_mesh,
    scratch_types=[
        pltpu.VMEM_SHARED((N_SEG, COLS), jnp.float32),
        pltpu.VMEM((N_SEG, COLS), jnp.float32),
        pltpu.VMEM((NBUF, CHUNK, COLS), jnp.float32),
        pltpu.VMEM((NBUF, CHUNK), jnp.int32),
        pltpu.VMEM((N_SEG // CHUNK, CHUNK), jnp.int32),
    ] + [pltpu.SemaphoreType.DMA] * (2 * NBUF),
    compiler_params=pltpu.CompilerParams(use_tc_tiling_on_sc=False,
                                         needs_layout_passes=False),
)
def _sc_segment_sum(x_hbm, b_hbm, out_hbm, acc_sp, acc_t, xbuf, idxbuf,
                    idbuf, *sems):
    semx = sems[0:NBUF]
    semi = sems[NBUF:2 * NBUF]
    c = lax.axis_index("c")
    s = lax.axis_index("s")
    col0 = c * COLS
    base = s * ROWS_MAIN

    zvec = jnp.zeros((16,), jnp.float32)
    def _zacc(i, carry):
        for q in range(NQ):
            acc_t[i, 16 * q:16 * q + 16] = zvec
        return carry
    lax.fori_loop(0, N_SEG, _zacc, 0)
    pltpu.sync_copy(acc_t.at[pl.ds(0, SEG_PER_TILE)],
                    acc_sp.at[pl.ds(s * SEG_PER_TILE, SEG_PER_TILE)])
    lane = lax.iota(jnp.int32, 16)
    for r in range(N_SEG // CHUNK):
        for q in range(CHUNK // 16):
            idbuf[r, 16 * q:16 * q + 16] = lane + (r * CHUNK + 16 * q)
    plsc.subcore_barrier()

    def load_descs(jj, b):
        start = base + jj * CHUNK
        return (
            pltpu.make_async_copy(b_hbm.at[pl.ds(start, CHUNK)],
                                  idxbuf.at[b], semi[b]),
            pltpu.make_async_copy(
                x_hbm.at[pl.ds(start, CHUNK), pl.ds(col0, COLS)],
                xbuf.at[b], semx[b]),
        )

    lane16 = lax.iota(jnp.int32, 16)

    def window_op(b, w):
        g0 = GROUP * w
        idv = idxbuf[b, pl.ds(g0, GROUP)]
        id_first = idv[0]
        id_last = idv[GROUP - 1]

        @pl.when(id_first == id_last)
        def _():
            for q in range(NQ):
                ssum = _tree_sum([xbuf[b, g0 + i, 16 * q:16 * q + 16]
                                  for i in range(GROUP)])
                plsc.addupdate(acc_t.at[id_first, pl.ds(16 * q, 16)], ssum)

        @pl.when(id_first != id_last)
        def _():
            rows = lane16 + g0
            bvec = jnp.full((16,), b, jnp.int32)
            for ccol in range(COLS):
                cvec = jnp.full((16,), ccol, jnp.int32)
                vals = plsc.load_gather(xbuf, [bvec, rows, cvec])
                plsc.addupdate_scatter(acc_t, [idv, cvec], vals)

    def compute_chunk(b):
        def _w(w, carry):
            window_op(b, w)
            return carry
        lax.fori_loop(0, CHUNK // GROUP, _w, 0)

    for b in range(3):
        for d in load_descs(b, b):
            d.start()

    def pipe(j, carry):
        for b in range(NBUF):
            jj = NBUF * j + b
            for d in load_descs(jj, b):
                d.wait()

            @pl.when(jj + 3 < NFULL)
            def _():
                for d in load_descs(jj + 3, (b + 3) % NBUF):
                    d.start()

            compute_chunk(b)
        return carry
    lax.fori_loop(0, NFULL // NBUF, pipe, 0)

    def rem_load(nrows):
        start = base + NFULL * CHUNK
        pltpu.sync_copy(b_hbm.at[pl.ds(start, nrows)],
                        idxbuf.at[0, pl.ds(0, nrows)])
        pltpu.sync_copy(x_hbm.at[pl.ds(start, nrows), pl.ds(col0, COLS)],
                        xbuf.at[0, pl.ds(0, nrows)])

    @pl.when(s < NS - 1)
    def _():
        rem_load(REM_MAIN)

    @pl.when(s == NS - 1)
    def _():
        rem_load(REM_LAST)

    n_rem_w = jnp.where(s == NS - 1, REM_LAST // GROUP, REM_MAIN // GROUP)

    def _rw(w, carry):
        window_op(0, w)
        return carry
    lax.fori_loop(0, n_rem_w, _rw, 0)

    for r in range(N_SEG // CHUNK):
        pltpu.sync_copy(acc_t.at[pl.ds(r * CHUNK, CHUNK)],
                        acc_sp.at[idbuf.at[r]], add=True)

    plsc.subcore_barrier()
    pltpu.sync_copy(acc_sp.at[pl.ds(s * SEG_PER_TILE, SEG_PER_TILE)],
                    out_hbm.at[pl.ds(s * SEG_PER_TILE, SEG_PER_TILE),
                               pl.ds(col0, COLS)])


def kernel(x, batch):
    return _sc_segment_sum(x, batch.astype(jnp.int32))
